# Initial kernel scaffold; baseline (speedup 1.0000x reference)
import jax
import jax.numpy as jnp
from jax.experimental import pallas as pl


def kernel(x, orig_bev, selected_indices, ego_index):
    raise NotImplementedError("write your pallas kernel here")


# speedup vs baseline: 12.2740x; 12.2740x over previous
import functools

import jax
import jax.numpy as jnp
from jax import lax
from jax.experimental import pallas as pl
from jax.experimental.pallas import tpu as pltpu

_KB = 2048
_BW = 2048


def _mask_kernel(idx_lane_ref, idx_sub_ref, mask_ref, *, w):
    il = idx_lane_ref[0]
    isub = idx_sub_ref[0]
    h = mask_ref.shape[0]
    kb = il.shape[1]
    p = (il // w == lax.broadcasted_iota(jnp.int32, (h, kb), 0)
         ).astype(jnp.bfloat16)
    q = (isub % w == lax.broadcasted_iota(jnp.int32, (kb, w), 1)
         ).astype(jnp.bfloat16)
    part = jnp.dot(p, q, preferred_element_type=jnp.float32)

    k = pl.program_id(0)

    @pl.when(k == 0)
    def _():
        mask_ref[...] = part

    @pl.when(k != 0)
    def _():
        mask_ref[...] += part


def _select_kernel(mask_ref, x_ref, ego_ref, out_ref):
    sel = mask_ref[...] > 0.5
    out_ref[...] = jnp.where(sel[None], x_ref[...], ego_ref[...][None])


def kernel(x, orig_bev, selected_indices, ego_index):
    n, c, h, w = orig_bev.shape
    hw = h * w
    k = selected_indices.shape[0]

    idx = selected_indices.astype(jnp.int32)
    idx_lane = idx.reshape(k // _KB, 1, _KB)
    idx_sub = idx.reshape(k // _KB, _KB, 1)

    mask2d = pl.pallas_call(
        functools.partial(_mask_kernel, w=w),
        grid=(k // _KB,),
        in_specs=[
            pl.BlockSpec((1, 1, _KB), lambda i: (i, 0, 0)),
            pl.BlockSpec((1, _KB, 1), lambda i: (i, 0, 0)),
        ],
        out_specs=pl.BlockSpec((h, w), lambda i: (0, 0)),
        out_shape=jax.ShapeDtypeStruct((h, w), jnp.float32),
        compiler_params=pltpu.CompilerParams(
            dimension_semantics=("arbitrary",)),
        name="bevt_mask",
    )(idx_lane, idx_sub)
    maskf = mask2d.reshape(1, hw)

    x3 = x.reshape(x.shape[1], c, hw)[:n]
    ego = lax.dynamic_index_in_dim(
        orig_bev, ego_index, axis=0, keepdims=False).reshape(c, hw)

    out = pl.pallas_call(
        _select_kernel,
        grid=(hw // _BW,),
        in_specs=[
            pl.BlockSpec((1, _BW), lambda i: (0, i)),
            pl.BlockSpec((n, c, _BW), lambda i: (0, 0, i)),
            pl.BlockSpec((c, _BW), lambda i: (0, i)),
        ],
        out_specs=pl.BlockSpec((n, c, _BW), lambda i: (0, 0, i)),
        out_shape=jax.ShapeDtypeStruct((n, c, hw), jnp.float32),
        compiler_params=pltpu.CompilerParams(
            dimension_semantics=("parallel",)),
        name="bevt_select",
    )(maskf, x3, ego)
    return out.reshape(n, c, h, w)


# speedup vs baseline: 25.3170x; 2.0627x over previous
import functools

import jax
import jax.numpy as jnp
from jax import lax
from jax.experimental import pallas as pl
from jax.experimental.pallas import tpu as pltpu

_KB = 2048
_BW = 2048


def _mask_kernel(idx_lane_ref, idx_sub_ref, mask_ref, *, w):
    il = idx_lane_ref[0]
    isub = idx_sub_ref[0]
    h = mask_ref.shape[0]
    kb = il.shape[1]
    p = (il // w == lax.broadcasted_iota(jnp.int32, (h, kb), 0)
         ).astype(jnp.bfloat16)
    q = (isub % w == lax.broadcasted_iota(jnp.int32, (kb, w), 1)
         ).astype(jnp.bfloat16)
    part = jnp.dot(p, q, preferred_element_type=jnp.float32)

    k = pl.program_id(0)

    @pl.when(k == 0)
    def _():
        mask_ref[...] = part

    @pl.when(k != 0)
    def _():
        mask_ref[...] += part


def _select_kernel(ego_idx_ref, mask_ref, x_ref, ego_ref, out_ref):
    del ego_idx_ref
    n, c, hb, w = out_ref.shape
    sel = mask_ref[...] > 0.5
    xb = x_ref[...].reshape(n, c, hb, w)
    out_ref[...] = jnp.where(sel[None, None], xb, ego_ref[...])


def kernel(x, orig_bev, selected_indices, ego_index):
    n, c, h, w = orig_bev.shape
    hw = h * w
    k = selected_indices.shape[0]

    idx = selected_indices.astype(jnp.int32)
    idx_lane = idx.reshape(k // _KB, 1, _KB)
    idx_sub = idx.reshape(k // _KB, _KB, 1)

    mask2d = pl.pallas_call(
        functools.partial(_mask_kernel, w=w),
        grid=(k // _KB,),
        in_specs=[
            pl.BlockSpec((1, 1, _KB), lambda i: (i, 0, 0)),
            pl.BlockSpec((1, _KB, 1), lambda i: (i, 0, 0)),
        ],
        out_specs=pl.BlockSpec((h, w), lambda i: (0, 0)),
        out_shape=jax.ShapeDtypeStruct((h, w), jnp.float32),
        compiler_params=pltpu.CompilerParams(
            dimension_semantics=("arbitrary",)),
        name="bevt_mask",
    )(idx_lane, idx_sub)
    x3 = x.reshape(x.shape[1], c, hw)[:n]
    hb = _BW // w
    ego_i = jnp.where(ego_index < 0, ego_index + n, ego_index)
    ego_arr = jnp.asarray(ego_i, jnp.int32).reshape(1)

    out = pl.pallas_call(
        _select_kernel,
        grid_spec=pltpu.PrefetchScalarGridSpec(
            num_scalar_prefetch=1,
            grid=(h // hb,),
            in_specs=[
                pl.BlockSpec((hb, w), lambda i, e: (i, 0)),
                pl.BlockSpec((n, c, _BW), lambda i, e: (0, 0, i)),
                pl.BlockSpec((1, c, hb, w), lambda i, e: (e[0], 0, i, 0)),
            ],
            out_specs=pl.BlockSpec((n, c, hb, w), lambda i, e: (0, 0, i, 0)),
        ),
        out_shape=jax.ShapeDtypeStruct((n, c, h, w), jnp.float32),
        compiler_params=pltpu.CompilerParams(
            dimension_semantics=("parallel",)),
        name="bevt_select",
    )(ego_arr, mask2d, x3, orig_bev)
    return out


# speedup vs baseline: 26.4569x; 1.0450x over previous
import functools

import jax
import jax.numpy as jnp
from jax import lax
from jax.experimental import pallas as pl
from jax.experimental.pallas import tpu as pltpu

_KB = 4096
_BW = 4096


def _mask_kernel(idx_lane_ref, idx_sub_ref, mask_ref, *, w):
    il = idx_lane_ref[0]
    isub = idx_sub_ref[0]
    h = mask_ref.shape[0]
    kb = il.shape[1]
    p = (il // w == lax.broadcasted_iota(jnp.int32, (h, kb), 0)
         ).astype(jnp.bfloat16)
    q = (isub % w == lax.broadcasted_iota(jnp.int32, (kb, w), 1)
         ).astype(jnp.bfloat16)
    part = jnp.dot(p, q, preferred_element_type=jnp.float32)

    k = pl.program_id(0)

    @pl.when(k == 0)
    def _():
        mask_ref[...] = part

    @pl.when(k != 0)
    def _():
        mask_ref[...] += part


def _select_kernel(ego_idx_ref, mask_ref, x_ref, ego_ref, out_ref):
    del ego_idx_ref
    n, c, hb, w = out_ref.shape
    sel = mask_ref[...] > 0.5
    xb = x_ref[...].reshape(n, c, hb, w)
    out_ref[...] = jnp.where(sel[None, None], xb, ego_ref[...])


def kernel(x, orig_bev, selected_indices, ego_index):
    n, c, h, w = orig_bev.shape
    hw = h * w
    k = selected_indices.shape[0]

    idx = selected_indices.astype(jnp.int32)
    idx_lane = idx.reshape(k // _KB, 1, _KB)
    idx_sub = idx.reshape(k // _KB, _KB, 1)

    mask2d = pl.pallas_call(
        functools.partial(_mask_kernel, w=w),
        grid=(k // _KB,),
        in_specs=[
            pl.BlockSpec((1, 1, _KB), lambda i: (i, 0, 0)),
            pl.BlockSpec((1, _KB, 1), lambda i: (i, 0, 0)),
        ],
        out_specs=pl.BlockSpec((h, w), lambda i: (0, 0)),
        out_shape=jax.ShapeDtypeStruct((h, w), jnp.float32),
        compiler_params=pltpu.CompilerParams(
            dimension_semantics=("arbitrary",)),
        name="bevt_mask",
    )(idx_lane, idx_sub)
    x3 = x.reshape(x.shape[1], c, hw)[:n]
    hb = _BW // w
    ego_i = jnp.where(ego_index < 0, ego_index + n, ego_index)
    ego_arr = jnp.asarray(ego_i, jnp.int32).reshape(1)

    out = pl.pallas_call(
        _select_kernel,
        grid_spec=pltpu.PrefetchScalarGridSpec(
            num_scalar_prefetch=1,
            grid=(h // hb,),
            in_specs=[
                pl.BlockSpec((hb, w), lambda i, e: (i, 0)),
                pl.BlockSpec((n, c, _BW), lambda i, e: (0, 0, i)),
                pl.BlockSpec((1, c, hb, w), lambda i, e: (e[0], 0, i, 0)),
            ],
            out_specs=pl.BlockSpec((n, c, hb, w), lambda i, e: (0, 0, i, 0)),
        ),
        out_shape=jax.ShapeDtypeStruct((n, c, h, w), jnp.float32),
        compiler_params=pltpu.CompilerParams(
            dimension_semantics=("parallel",)),
        name="bevt_select",
    )(ego_arr, mask2d, x3, orig_bev)
    return out


# speedup vs baseline: 32.4185x; 1.2253x over previous
import functools

import jax
import jax.numpy as jnp
from jax import lax
from jax.experimental import pallas as pl
from jax.experimental.pallas import tpu as pltpu

_KB = 4096
_HB = 16


def _fused_kernel(ego_idx_ref, idx_lane_ref, x_ref, ego_ref,
                  out_ref, mask_scr):
    del ego_idx_ref
    n, c, hb, w = out_ref.shape
    h = mask_scr.shape[0]
    k = idx_lane_ref.shape[1]
    i = pl.program_id(0)

    @pl.when(i == 0)
    def _():
        acc = jnp.zeros((h, w), jnp.float32)
        for kb in range(k // _KB):
            sl = pl.ds(kb * _KB, _KB)
            il = idx_lane_ref[:, sl]
            pt = (il // w == lax.broadcasted_iota(jnp.int32, (h, _KB), 0)
                  ).astype(jnp.bfloat16)
            qt = (il % w == lax.broadcasted_iota(jnp.int32, (w, _KB), 0)
                  ).astype(jnp.bfloat16)
            acc = acc + lax.dot_general(
                pt, qt, (((1,), (1,)), ((), ())),
                preferred_element_type=jnp.float32)
        mask_scr[...] = acc

    sel = mask_scr[pl.ds(i * hb, hb), :] > 0.5
    xb = x_ref[...].reshape(n, c, hb, w)
    out_ref[...] = jnp.where(sel[None, None], xb, ego_ref[...])


def kernel(x, orig_bev, selected_indices, ego_index):
    n, c, h, w = orig_bev.shape
    hw = h * w
    k = selected_indices.shape[0]

    idx_lane = selected_indices.astype(jnp.int32).reshape(1, k)

    x3 = x.reshape(x.shape[1], c, hw)[:n]
    bw = _HB * w
    ego_i = jnp.where(ego_index < 0, ego_index + n, ego_index)
    ego_arr = jnp.asarray(ego_i, jnp.int32).reshape(1)

    out = pl.pallas_call(
        functools.partial(_fused_kernel),
        grid_spec=pltpu.PrefetchScalarGridSpec(
            num_scalar_prefetch=1,
            grid=(h // _HB,),
            in_specs=[
                pl.BlockSpec((1, k), lambda i, e: (0, 0)),
                pl.BlockSpec((n, c, bw), lambda i, e: (0, 0, i)),
                pl.BlockSpec((1, c, _HB, w), lambda i, e: (e[0], 0, i, 0)),
            ],
            out_specs=pl.BlockSpec((n, c, _HB, w), lambda i, e: (0, 0, i, 0)),
            scratch_shapes=[pltpu.VMEM((h, w), jnp.float32)],
        ),
        out_shape=jax.ShapeDtypeStruct((n, c, h, w), jnp.float32),
        compiler_params=pltpu.CompilerParams(
            dimension_semantics=("arbitrary",)),
        name="bevt_fused",
    )(ego_arr, idx_lane, x3, orig_bev)
    return out


# speedup vs baseline: 32.4521x; 1.0010x over previous
import jax
import jax.numpy as jnp
from jax import lax
from jax.experimental import pallas as pl
from jax.experimental.pallas import tpu as pltpu

_KB = 4096
_HB = 16


def _fused_kernel(ego_idx_ref, idx_lane_ref, x_ref, ego_ref,
                  out_ref, mask_scr):
    del ego_idx_ref
    n, c, hb, w = out_ref.shape
    h = mask_scr.shape[0]
    k = idx_lane_ref.shape[1]
    i = pl.program_id(0)

    @pl.when(i == 0)
    def _():
        acc = jnp.zeros((h, w), jnp.float32)
        for kb in range(k // _KB):
            sl = pl.ds(kb * _KB, _KB)
            il = idx_lane_ref[:, sl]
            pt = (il // w == lax.broadcasted_iota(jnp.int32, (h, _KB), 0)
                  ).astype(jnp.bfloat16)
            qt = (il % w == lax.broadcasted_iota(jnp.int32, (w, _KB), 0)
                  ).astype(jnp.bfloat16)
            acc = acc + lax.dot_general(
                pt, qt, (((1,), (1,)), ((), ())),
                preferred_element_type=jnp.float32)
        mask_scr[...] = acc

    sel = mask_scr[pl.ds(i * hb, hb), :] > 0.5
    xb = x_ref[...].reshape(n, c, hb, w)
    out_ref[...] = jnp.where(sel[None, None], xb, ego_ref[...])


def kernel(x, orig_bev, selected_indices, ego_index):
    n, c, h, w = orig_bev.shape
    hw = h * w
    k = selected_indices.shape[0]

    idx_lane = selected_indices.astype(jnp.int32).reshape(1, k)

    x3 = x.reshape(x.shape[1], c, hw)[:n]
    bw = _HB * w
    ego_i = jnp.where(ego_index < 0, ego_index + n, ego_index)
    ego_arr = jnp.asarray(ego_i, jnp.int32).reshape(1)

    out = pl.pallas_call(
        _fused_kernel,
        grid_spec=pltpu.PrefetchScalarGridSpec(
            num_scalar_prefetch=1,
            grid=(h // _HB,),
            in_specs=[
                pl.BlockSpec((1, k), lambda i, e: (0, 0)),
                pl.BlockSpec((n, c, bw), lambda i, e: (0, 0, i)),
                pl.BlockSpec((1, c, _HB, w), lambda i, e: (e[0], 0, i, 0)),
            ],
            out_specs=pl.BlockSpec((n, c, _HB, w), lambda i, e: (0, 0, i, 0)),
            scratch_shapes=[pltpu.VMEM((h, w), jnp.float32)],
        ),
        out_shape=jax.ShapeDtypeStruct((n, c, h, w), jnp.float32),
        compiler_params=pltpu.CompilerParams(
            dimension_semantics=("arbitrary",)),
        name="bevt_fused",
    )(ego_arr, idx_lane, x3, orig_bev)
    return out
